# baseline (speedup 1.0000x reference)
import jax
import jax.numpy as jnp
from jax.experimental import pallas as pl
from jax.experimental.pallas import tpu as pltpu

_NUM_CLASSES = 1000


def _linear_kernel(x_ref, wt_ref, b_ref, o_ref, wbf_ref):
    @pl.when(pl.program_id(0) == 0)
    def _():
        wbf_ref[...] = wt_ref[...].astype(jnp.bfloat16)

    x = x_ref[...].astype(jnp.bfloat16)
    acc = jnp.dot(x, wbf_ref[...], preferred_element_type=jnp.float32)
    o_ref[...] = (acc + b_ref[...]).astype(jnp.bfloat16)


def kernel(x, wt_p, b_p):
    M, K = x.shape
    K_pad, N_pad = wt_p.shape
    n = min(_NUM_CLASSES, N_pad)

    tile_m = next(t for t in (1024, 512, 256, 128, 64, 8, 1) if M % t == 0)
    m_steps = M // tile_m

    cost = pl.CostEstimate(
        flops=2 * M * K_pad * N_pad,
        transcendentals=0,
        bytes_accessed=M * K * 4 + K_pad * N_pad * 4 + N_pad * 4 + M * N_pad * 2,
    )

    out_bf = pl.pallas_call(
        _linear_kernel,
        out_shape=jax.ShapeDtypeStruct((M, N_pad), jnp.bfloat16),
        grid=(m_steps,),
        in_specs=[
            pl.BlockSpec((tile_m, K), lambda i: (i, 0)),
            pl.BlockSpec((K_pad, N_pad), lambda i: (0, 0)),
            pl.BlockSpec((1, N_pad), lambda i: (0, 0)),
        ],
        out_specs=pl.BlockSpec((tile_m, N_pad), lambda i: (i, 0)),
        scratch_shapes=[pltpu.VMEM((K_pad, N_pad), jnp.bfloat16)],
        compiler_params=pltpu.CompilerParams(
            dimension_semantics=("arbitrary",),
        ),
        cost_estimate=cost,
    )(x, wt_p, b_p)
    return out_bf[:, :n].astype(x.dtype)


# speedup vs baseline: 1.3534x; 1.3534x over previous
import jax
import jax.numpy as jnp
from jax.experimental import pallas as pl
from jax.experimental.pallas import tpu as pltpu

_NUM_CLASSES = 1000
_SPLIT = 2


def _out_copies(o_ref, acc_ref, tail_ref, sem_ref, t, tile_m, n_al, n):
    s = jax.lax.rem(t, 2)
    rows_pp = tile_m // _SPLIT
    copies = []
    for p in range(_SPLIT):
        copies.append((
            pltpu.make_async_copy(
                acc_ref.at[s, pl.ds(p * rows_pp, rows_pp)],
                o_ref.at[pl.ds(t * tile_m + p * rows_pp, rows_pp),
                         pl.ds(0, n_al)],
                sem_ref.at[s, p],
            ),
            p,
        ))
    if n > n_al:
        copies.append((
            pltpu.make_async_copy(
                tail_ref.at[s],
                o_ref.at[pl.ds(t * tile_m, tile_m), pl.ds(n_al, n - n_al)],
                sem_ref.at[s, _SPLIT],
            ),
            1,
        ))
    return copies


def _linear_kernel(x_ref, wt_ref, b_ref, o_ref, wbf_ref, acc_ref, tail_ref,
                   sem_ref):
    i = pl.program_id(0)
    nsteps = pl.num_programs(0)
    slot = jax.lax.rem(i, 2)
    tile_m = x_ref.shape[0]
    n = o_ref.shape[1]
    n_al = (n // 128) * 128

    @pl.when(i == 0)
    def _():
        wbf_ref[...] = wt_ref[...].astype(jnp.bfloat16)

    @pl.when(i >= 2)
    def _():
        for c, _ in _out_copies(o_ref, acc_ref, tail_ref, sem_ref, i - 2,
                                tile_m, n_al, n):
            c.wait()

    x = x_ref[...].astype(jnp.bfloat16)
    acc = jnp.dot(x, wbf_ref[...], preferred_element_type=jnp.float32)
    acc = acc + b_ref[...]
    acc_ref[slot] = acc[:, :n_al]
    if n > n_al:
        tail_ref[slot] = acc[:, n_al:n]

    for c, p in _out_copies(o_ref, acc_ref, tail_ref, sem_ref, i,
                            tile_m, n_al, n):
        c.start(priority=p)

    @pl.when(i == nsteps - 1)
    def _():
        @pl.when(nsteps >= 2)
        def _():
            for c, _ in _out_copies(o_ref, acc_ref, tail_ref, sem_ref, i - 1,
                                    tile_m, n_al, n):
                c.wait()

        for c, _ in _out_copies(o_ref, acc_ref, tail_ref, sem_ref, i,
                                tile_m, n_al, n):
            c.wait()


def kernel(x, wt_p, b_p):
    M, K = x.shape
    K_pad, N_pad = wt_p.shape
    n = min(_NUM_CLASSES, N_pad)
    n_al = (n // 128) * 128
    n_tail = max(n - n_al, 8)

    tile_m = next(t for t in (1024, 512, 256, 128, 64, 8, 1) if M % t == 0)
    m_steps = M // tile_m

    cost = pl.CostEstimate(
        flops=2 * M * K_pad * N_pad,
        transcendentals=0,
        bytes_accessed=M * K * 4 + K_pad * N_pad * 4 + N_pad * 4 + M * n * 4,
    )

    return pl.pallas_call(
        _linear_kernel,
        out_shape=jax.ShapeDtypeStruct((M, n), x.dtype),
        grid=(m_steps,),
        in_specs=[
            pl.BlockSpec((tile_m, K), lambda i: (i, 0)),
            pl.BlockSpec((K_pad, N_pad), lambda i: (0, 0)),
            pl.BlockSpec((1, N_pad), lambda i: (0, 0)),
        ],
        out_specs=pl.BlockSpec(memory_space=pl.ANY),
        scratch_shapes=[
            pltpu.VMEM((K_pad, N_pad), jnp.bfloat16),
            pltpu.VMEM((2, tile_m, n_al), jnp.float32),
            pltpu.VMEM((2, tile_m, n_tail), jnp.float32),
            pltpu.SemaphoreType.DMA((2, _SPLIT + 1)),
        ],
        compiler_params=pltpu.CompilerParams(
            dimension_semantics=("arbitrary",),
        ),
        cost_estimate=cost,
    )(x, wt_p, b_p)
